# baseline (speedup 1.0000x reference)
import functools

import jax
import jax.numpy as jnp
from jax.experimental import pallas as pl
from jax.experimental.pallas import tpu as pltpu

_BF = jnp.bfloat16
_F32 = jnp.float32
_F8 = jnp.float8_e4m3fn
_TN = (((0,), (0,)), ((), ()))
_NT = (((1,), (1,)), ((), ()))


def _gradg_body(x_ref, wkvt_hbm, m1t_hbm, m2t_hbm, m2_hbm, wgt_hbm, hbg_ref,
                gs_ref, g1_hbm, g2_hbm,
                wkvt, m1t, m2t, m2, wgt, ka, aa, ra, dha,
                g1_ref, g2_ref, sems):
    i = pl.program_id(0)
    j = pl.program_id(1)
    nb = pl.num_programs(1)
    d = x_ref.shape[1]
    tn = x_ref.shape[0]
    sub = tn // 2

    @pl.when(j == 0)
    def _():
        c0 = pltpu.make_async_copy(wkvt_hbm, wkvt, sems.at[0])
        c1 = pltpu.make_async_copy(m1t_hbm, m1t, sems.at[1])
        c2 = pltpu.make_async_copy(m2t_hbm, m2t, sems.at[2])
        c3 = pltpu.make_async_copy(m2_hbm, m2, sems.at[3])
        c4 = pltpu.make_async_copy(wgt_hbm, wgt, sems.at[4])
        c0.start(); c1.start(); c2.start(); c3.start(); c4.start()
        c0.wait(); c1.wait(); c2.wait(); c3.wait(); c4.wait()
        g1_ref[...] = jnp.zeros_like(g1_ref)
        g2_ref[...] = jnp.zeros_like(g2_ref)
        gs_ref[...] = jnp.zeros_like(gs_ref)

    for p in range(2):
        sl = slice(p * sub, (p + 1) * sub)
        xs = x_ref[sl, :].astype(_BF)
        gg = jnp.dot(xs.astype(_F8), wgt[...], preferred_element_type=_F32)
        t = jnp.tanh(0.5 * gg + hbg_ref[...])
        c = t.shape[1]
        gs_ref[0] += jnp.sum(t.reshape(sub // 8, 8, c), axis=0)

        kv = jnp.dot(xs, wkvt[...], preferred_element_type=_F32)
        k = kv[:, :d].astype(_BF)
        v = kv[:, d:]
        ka[sl, :] = k
        h = jnp.dot(k, m1t[...], preferred_element_type=_F32)
        sig = 0.5 * (1.0 + jnp.tanh(0.5 * h))
        a = h * sig
        a_bf = a.astype(_BF)
        aa[sl, :] = a_bf
        pred = jnp.dot(a_bf, m2t[...], preferred_element_type=_F32)
        r_bf = ((pred - v) * (2.0 / d)).astype(_BF)
        ra[sl, :] = r_bf
        da = jnp.dot(r_bf, m2[...], preferred_element_type=_F32)
        dha[sl, :] = (da * (sig * (1.0 + h * (1.0 - sig)))).astype(_BF)

    g1_ref[...] += jax.lax.dot_general(dha[...], ka[...], _TN,
                                       preferred_element_type=_F32)
    g2_ref[...] += jax.lax.dot_general(ra[...], aa[...], _TN,
                                       preferred_element_type=_F32)

    @pl.when(j == nb - 1)
    def _():
        c5 = pltpu.make_async_copy(g1_ref, g1_hbm.at[i], sems.at[0])
        c6 = pltpu.make_async_copy(g2_ref, g2_hbm.at[i], sems.at[1])
        c5.start(); c6.start()
        c5.wait(); c6.wait()


def _update_body(gs_ref, m1_ref, s1_ref, g1a_ref, g1b_ref,
                 m2_ref, s2_ref, g2a_ref, g2b_ref,
                 out1_ref, out2_ref, nd):
    t = gs_ref[...]
    d = t.shape[2] // 3
    alpha = 0.5 + 0.5 * jnp.sum(t[:, :, :d]) / nd
    theta = 0.5 + 0.5 * jnp.sum(t[:, :, d:2 * d]) / nd
    eta = 0.5 + 0.5 * jnp.sum(t[:, :, 2 * d:]) / nd
    out1_ref[...] = ((1.0 - alpha) * m1_ref[...] + eta * s1_ref[...]
                     - theta * (g1a_ref[0] + g1b_ref[0])).astype(_BF)
    out2_ref[...] = ((1.0 - alpha) * m2_ref[...] + eta * s2_ref[...]
                     - theta * (g2a_ref[0] + g2b_ref[0])).astype(_BF)


def _retr_body(x_ref, wqt_ref, woutt_ref, m1nt_hbm, m2nt_hbm, out_ref,
               m1nt, m2nt, sems):
    j = pl.program_id(1)

    @pl.when(j == 0)
    def _():
        c0 = pltpu.make_async_copy(m1nt_hbm, m1nt, sems.at[0])
        c1 = pltpu.make_async_copy(m2nt_hbm, m2nt, sems.at[1])
        c0.start(); c1.start()
        c0.wait(); c1.wait()

    half = x_ref.shape[0] // 2
    for p in range(2):
        sl = slice(p * half, (p + 1) * half)
        q = jax.lax.dot_general(x_ref[sl, :].astype(_BF), wqt_ref[...], _NT,
                                preferred_element_type=_F32).astype(_BF)
        hq = jax.lax.dot_general(q, m1nt[...], _NT,
                                 preferred_element_type=_F32)
        aq = (hq * (0.5 * (1.0 + jnp.tanh(0.5 * hq)))).astype(_BF)
        retr = jax.lax.dot_general(aq, m2nt[...], _NT,
                                   preferred_element_type=_F32).astype(_BF)
        out_ref[sl, :] = jax.lax.dot_general(retr, woutt_ref[...], _NT,
                                             preferred_element_type=_F32)


def kernel(x, Wk, Wv, Wq, Wout, Wgd, bgd, Wgl, bgl, Wgm, bgm, M1, M2, S1, S2):
    b, s, d = x.shape
    h = M1.shape[0]
    n = b * s
    xf = x.reshape(n, d)

    ncores = 2
    vmem = pltpu.CompilerParams(
        dimension_semantics=("parallel", "arbitrary"),
        vmem_limit_bytes=58 * 1024 * 1024,
    )

    wgt = jnp.concatenate([Wgd, Wgl, Wgm], axis=0).astype(_F8).T
    hbg = 0.5 * jnp.concatenate([bgd, bgl, bgm]).reshape(1, 3 * d)
    wkvt = jnp.concatenate([Wk, Wv], axis=0).astype(_BF).T
    m1t_bf = M1.astype(_BF).T
    m2t_bf = M2.astype(_BF).T
    m2_bf = M2.astype(_BF)

    tn = min(512, n // ncores)
    nb = n // (ncores * tn)
    gate_sums, g1p, g2p = pl.pallas_call(
        _gradg_body,
        grid=(ncores, nb),
        in_specs=[
            pl.BlockSpec((tn, d), lambda i, j: (i * nb + j, 0)),
            pl.BlockSpec(memory_space=pl.ANY),
            pl.BlockSpec(memory_space=pl.ANY),
            pl.BlockSpec(memory_space=pl.ANY),
            pl.BlockSpec(memory_space=pl.ANY),
            pl.BlockSpec(memory_space=pl.ANY),
            pl.BlockSpec((1, 3 * d), lambda i, j: (0, 0)),
        ],
        out_specs=[
            pl.BlockSpec((1, 8, 3 * d), lambda i, j: (i, 0, 0)),
            pl.BlockSpec(memory_space=pl.ANY),
            pl.BlockSpec(memory_space=pl.ANY),
        ],
        out_shape=[
            jax.ShapeDtypeStruct((ncores, 8, 3 * d), _F32),
            jax.ShapeDtypeStruct((ncores, h, d), _F32),
            jax.ShapeDtypeStruct((ncores, d, h), _F32),
        ],
        scratch_shapes=[
            pltpu.VMEM((d, 2 * d), _BF),
            pltpu.VMEM((d, h), _BF),
            pltpu.VMEM((h, d), _BF),
            pltpu.VMEM((d, h), _BF),
            pltpu.VMEM((d, 3 * d), _F8),
            pltpu.VMEM((tn, d), _BF),
            pltpu.VMEM((tn, h), _BF),
            pltpu.VMEM((tn, d), _BF),
            pltpu.VMEM((tn, h), _BF),
            pltpu.VMEM((h, d), _F32),
            pltpu.VMEM((d, h), _F32),
            pltpu.SemaphoreType.DMA((5,)),
        ],
        compiler_params=vmem,
        name="ltm_gradg",
    )(xf, wkvt, m1t_bf, m2t_bf, m2_bf, wgt, hbg)

    rb = 8
    m1n, m2n = pl.pallas_call(
        functools.partial(_update_body, nd=float(n * d)),
        grid=(rb,),
        in_specs=[
            pl.BlockSpec((ncores, 8, 3 * d), lambda i: (0, 0, 0)),
            pl.BlockSpec((h // rb, d), lambda i: (i, 0)),
            pl.BlockSpec((h // rb, d), lambda i: (i, 0)),
            pl.BlockSpec((1, h // rb, d), lambda i: (0, i, 0)),
            pl.BlockSpec((1, h // rb, d), lambda i: (1, i, 0)),
            pl.BlockSpec((d // rb, h), lambda i: (i, 0)),
            pl.BlockSpec((d // rb, h), lambda i: (i, 0)),
            pl.BlockSpec((1, d // rb, h), lambda i: (0, i, 0)),
            pl.BlockSpec((1, d // rb, h), lambda i: (1, i, 0)),
        ],
        out_specs=[
            pl.BlockSpec((h // rb, d), lambda i: (i, 0)),
            pl.BlockSpec((d // rb, h), lambda i: (i, 0)),
        ],
        out_shape=[
            jax.ShapeDtypeStruct((h, d), _BF),
            jax.ShapeDtypeStruct((d, h), _BF),
        ],
        compiler_params=pltpu.CompilerParams(
            dimension_semantics=("parallel",),
        ),
        name="ltm_update",
    )(gate_sums, M1, S1, g1p, g1p, M2, S2, g2p, g2p)

    wqt = Wq.astype(_BF)
    woutt = Wout.astype(_BF)
    m1nt = m1n
    m2nt = m2n
    tnr = min(1024, n // ncores)
    nbr = n // (ncores * tnr)
    out = pl.pallas_call(
        _retr_body,
        grid=(ncores, nbr),
        in_specs=[
            pl.BlockSpec((tnr, d), lambda i, j: (i * nbr + j, 0)),
            pl.BlockSpec((d, d), lambda i, j: (0, 0)),
            pl.BlockSpec((d, d), lambda i, j: (0, 0)),
            pl.BlockSpec(memory_space=pl.ANY),
            pl.BlockSpec(memory_space=pl.ANY),
        ],
        out_specs=pl.BlockSpec((tnr, d), lambda i, j: (i * nbr + j, 0)),
        out_shape=jax.ShapeDtypeStruct((n, d), _F32),
        scratch_shapes=[
            pltpu.VMEM((h, d), _BF),
            pltpu.VMEM((d, h), _BF),
            pltpu.SemaphoreType.DMA((2,)),
        ],
        compiler_params=vmem,
        name="ltm_retrieve",
    )(xf, wqt, woutt, m1nt, m2nt)

    return out.reshape(b, s, d)
